# baseline (speedup 1.0000x reference)
import functools

import jax
import jax.numpy as jnp
from jax import lax
from jax.experimental import pallas as pl
from jax.experimental.pallas import tpu as pltpu
from jax.experimental.pallas import tpu_sc as plsc

_EMB_DIM = 64
_HALF = 32
_BATCH = 16384
_ENT = 1000000
_NC = 2
_NS = 16
_NW = _NC * _NS
_B_PER_W = _BATCH // _NW


def _gather_body(ent_flat, rel_hbm, h_idx_hbm, t_idx_hbm, r_idx_hbm,
                 h_out, t_out, r_out,
                 h_iv, t_iv, r_iv, fidx, stage, r_rows, sem, sem_r):
    wid = lax.axis_index("s") * _NC + lax.axis_index("c")
    base = wid * _B_PER_W
    pltpu.sync_copy(h_idx_hbm.at[pl.ds(base, _B_PER_W)], h_iv)
    pltpu.sync_copy(t_idx_hbm.at[pl.ds(base, _B_PER_W)], t_iv)
    pltpu.sync_copy(r_idx_hbm.at[pl.ds(base, _B_PER_W)], r_iv)
    cr = pltpu.async_copy(rel_hbm.at[r_iv], r_rows, sem_r)

    for iv, out in ((h_iv, h_out), (t_iv, t_out)):
        def off_step(g, carry, _iv=iv):
            gs = pl.ds(g * 16, 16)
            v = _iv[gs]
            for d in range(_EMB_DIM):
                fidx[d, gs] = v + jnp.int32(d * _ENT)
            return carry

        lax.fori_loop(0, _B_PER_W // 16, off_step, 0)
        handles = [
            pltpu.async_copy(ent_flat.at[fidx.at[d]], stage.at[d], sem)
            for d in range(_EMB_DIM)
        ]
        for h in handles:
            h.wait()
        pltpu.sync_copy(stage, out.at[:, pl.ds(base, _B_PER_W)])

    cr.wait()

    def r_step(g, carry):
        i16 = jax.lax.broadcasted_iota(jnp.int32, (16,), 0) + g * 16
        for d in range(_EMB_DIM):
            d16 = jnp.full((16,), d, jnp.int32)
            stage[d, pl.ds(g * 16, 16)] = plsc.load_gather(r_rows, [i16, d16])
        return carry

    lax.fori_loop(0, _B_PER_W // 16, r_step, 0)
    pltpu.sync_copy(stage, r_out.at[:, pl.ds(base, _B_PER_W)])


_gather3 = functools.partial(
    pl.kernel,
    out_type=[jax.ShapeDtypeStruct((_EMB_DIM, _BATCH), jnp.float32)] * 3,
    mesh=plsc.VectorSubcoreMesh(core_axis_name="c", subcore_axis_name="s"),
    scratch_types=[
        pltpu.VMEM((_B_PER_W,), jnp.int32),
        pltpu.VMEM((_B_PER_W,), jnp.int32),
        pltpu.VMEM((_B_PER_W,), jnp.int32),
        pltpu.VMEM((_EMB_DIM, _B_PER_W), jnp.int32),
        pltpu.VMEM((_EMB_DIM, _B_PER_W), jnp.float32),
        pltpu.VMEM((_B_PER_W, _EMB_DIM), jnp.float32),
        pltpu.SemaphoreType.DMA,
        pltpu.SemaphoreType.DMA,
    ],
    compiler_params=pltpu.CompilerParams(
        use_tc_tiling_on_sc=False, needs_layout_passes=False),
)(_gather_body)


_CCH = 4096
_NCC = _BATCH // _CCH


def _score_body(h_ref, t_ref, r_ref, o_ref):
    def ss_step(i, acc):
        s = pl.ds(i * _CCH, _CCH)
        hh = h_ref[:, s]
        tt = t_ref[:, s]
        rr = r_ref[:, s]
        hh = hh * hh
        tt = tt * tt
        rr = rr * rr
        return (acc[0] + jnp.sum(hh[:_HALF]), acc[1] + jnp.sum(hh[_HALF:]),
                acc[2] + jnp.sum(tt[:_HALF]), acc[3] + jnp.sum(tt[_HALF:]),
                acc[4] + jnp.sum(rr[:_HALF]), acc[5] + jnp.sum(rr[_HALF:]))

    z = jnp.float32(0)
    sh0, sh1, st0, st1, sr0, sr1 = lax.fori_loop(
        0, _NCC, ss_step, (z, z, z, z, z, z))
    d0 = jnp.sqrt(sh0) * jnp.sqrt(sr0) * jnp.sqrt(st0)
    d1 = jnp.sqrt(sh1) * jnp.sqrt(sr1) * jnp.sqrt(st1)

    def sc_step(i, carry):
        s = pl.ds(i * _CCH, _CCH)
        m = h_ref[:, s] * r_ref[:, s] * t_ref[:, s]
        acc = None
        for half, d in ((0, d0), (1, d1)):
            p = 1.0 - m[half * _HALF:(half + 1) * _HALF, :] / d
            w = _HALF
            while w > 1:
                w //= 2
                p = p[:w, :] * p[w:2 * w, :]
            score = 1.0 - p[0]
            acc = -score if acc is None else acc - score
        o_ref[s] = acc
        return carry

    lax.fori_loop(0, _NCC, sc_step, 0)


def kernel(ent_embeddings, rel_embeddings, predict_h, predict_t, predict_r):
    h_idx = predict_h.astype(jnp.int32)
    t_idx = predict_t.astype(jnp.int32)
    r_idx = predict_r.astype(jnp.int32)
    ent_flat = ent_embeddings.T.reshape(-1)
    hT, tT, rT = _gather3(
        ent_flat, rel_embeddings, h_idx, t_idx, r_idx)
    pred = pl.pallas_call(
        _score_body,
        out_shape=jax.ShapeDtypeStruct((_BATCH,), jnp.float32),
    )(hT, tT, rT)
    return pred


# speedup vs baseline: 7.4664x; 7.4664x over previous
import functools

import jax
import jax.numpy as jnp
from jax import lax
from jax.experimental import pallas as pl
from jax.experimental.pallas import tpu as pltpu
from jax.experimental.pallas import tpu_sc as plsc

_EMB_DIM = 64
_HALF = 32
_BATCH = 16384
_NC = 2
_NS = 16
_NW = _NC * _NS
_B_PER_W = _BATCH // _NW


def _gather_body(ent_hbm, rel_hbm, h_idx_hbm, t_idx_hbm, r_idx_hbm,
                 h_out, t_out, r_out,
                 h_iv, t_iv, r_iv, h_rows, t_rows, r_rows,
                 sem_h, sem_t, sem_r):
    wid = lax.axis_index("s") * _NC + lax.axis_index("c")
    base = wid * _B_PER_W
    pltpu.sync_copy(h_idx_hbm.at[pl.ds(base, _B_PER_W)], h_iv)
    pltpu.sync_copy(t_idx_hbm.at[pl.ds(base, _B_PER_W)], t_iv)
    pltpu.sync_copy(r_idx_hbm.at[pl.ds(base, _B_PER_W)], r_iv)
    ch = pltpu.async_copy(ent_hbm.at[h_iv], h_rows, sem_h)
    ct = pltpu.async_copy(ent_hbm.at[t_iv], t_rows, sem_t)
    cr = pltpu.async_copy(rel_hbm.at[r_iv], r_rows, sem_r)
    ch.wait()
    pltpu.sync_copy(h_rows, h_out.at[pl.ds(base, _B_PER_W)])
    ct.wait()
    pltpu.sync_copy(t_rows, t_out.at[pl.ds(base, _B_PER_W)])
    cr.wait()
    pltpu.sync_copy(r_rows, r_out.at[pl.ds(base, _B_PER_W)])


_gather3 = functools.partial(
    pl.kernel,
    out_type=[jax.ShapeDtypeStruct((_BATCH, _EMB_DIM), jnp.float32)] * 3,
    mesh=plsc.VectorSubcoreMesh(core_axis_name="c", subcore_axis_name="s"),
    scratch_types=[
        pltpu.VMEM((_B_PER_W,), jnp.int32),
        pltpu.VMEM((_B_PER_W,), jnp.int32),
        pltpu.VMEM((_B_PER_W,), jnp.int32),
        pltpu.VMEM((_B_PER_W, _EMB_DIM), jnp.float32),
        pltpu.VMEM((_B_PER_W, _EMB_DIM), jnp.float32),
        pltpu.VMEM((_B_PER_W, _EMB_DIM), jnp.float32),
        pltpu.SemaphoreType.DMA,
        pltpu.SemaphoreType.DMA,
        pltpu.SemaphoreType.DMA,
    ],
    compiler_params=pltpu.CompilerParams(use_tc_tiling_on_sc=False),
)(_gather_body)


_CHUNK = 2048
_NCHUNK = _BATCH // _CHUNK


def _score_body(h_ref, t_ref, r_ref, o_ref):
    def ss_step(i, acc):
        s = pl.ds(i * _CHUNK, _CHUNK)
        hh = h_ref[s, :]
        tt = t_ref[s, :]
        rr = r_ref[s, :]
        hh = hh * hh
        tt = tt * tt
        rr = rr * rr
        return (acc[0] + jnp.sum(hh[:, :_HALF]), acc[1] + jnp.sum(hh[:, _HALF:]),
                acc[2] + jnp.sum(tt[:, :_HALF]), acc[3] + jnp.sum(tt[:, _HALF:]),
                acc[4] + jnp.sum(rr[:, :_HALF]), acc[5] + jnp.sum(rr[:, _HALF:]))

    z = jnp.float32(0)
    sh0, sh1, st0, st1, sr0, sr1 = lax.fori_loop(
        0, _NCHUNK, ss_step, (z, z, z, z, z, z))
    d0 = jnp.sqrt(sh0) * jnp.sqrt(sr0) * jnp.sqrt(st0)
    d1 = jnp.sqrt(sh1) * jnp.sqrt(sr1) * jnp.sqrt(st1)

    def sc_step(i, carry):
        s = pl.ds(i * _CHUNK, _CHUNK)
        m = h_ref[s, :] * r_ref[s, :] * t_ref[s, :]
        acc = None
        for half, d in ((0, d0), (1, d1)):
            p = 1.0 - m[:, half * _HALF:(half + 1) * _HALF] / d
            w = _HALF
            while w > 1:
                w //= 2
                p = p[:, :w] * p[:, w:2 * w]
            score = 1.0 - p[:, 0]
            acc = -score if acc is None else acc - score
        o_ref[s] = acc
        return carry

    lax.fori_loop(0, _NCHUNK, sc_step, 0)


def kernel(ent_embeddings, rel_embeddings, predict_h, predict_t, predict_r):
    h_idx = predict_h.astype(jnp.int32)
    t_idx = predict_t.astype(jnp.int32)
    r_idx = predict_r.astype(jnp.int32)
    h_rows, t_rows, r_rows = _gather3(
        ent_embeddings, rel_embeddings, h_idx, t_idx, r_idx)
    pred = pl.pallas_call(
        _score_body,
        out_shape=jax.ShapeDtypeStruct((_BATCH,), jnp.float32),
    )(h_rows, t_rows, r_rows)
    return pred


# speedup vs baseline: 7.7424x; 1.0370x over previous
import functools

import jax
import jax.numpy as jnp
from jax import lax
from jax.experimental import pallas as pl
from jax.experimental.pallas import tpu as pltpu
from jax.experimental.pallas import tpu_sc as plsc

_EMB_DIM = 64
_HALF = 32
_BATCH = 16384
_NC = 2
_NS = 16
_NW = _NC * _NS
_B_PER_W = _BATCH // _NW


def _gather_body(ent_hbm, rel_hbm, h_idx_hbm, t_idx_hbm, r_idx_hbm,
                 h_out, t_out, r_out,
                 h_iv, t_iv, r_iv, h_rows, t_rows, r_rows,
                 sem_h, sem_t, sem_r):
    wid = lax.axis_index("s") * _NC + lax.axis_index("c")
    base = wid * _B_PER_W
    pltpu.sync_copy(h_idx_hbm.at[pl.ds(base, _B_PER_W)], h_iv)
    pltpu.sync_copy(t_idx_hbm.at[pl.ds(base, _B_PER_W)], t_iv)
    pltpu.sync_copy(r_idx_hbm.at[pl.ds(base, _B_PER_W)], r_iv)
    ch = pltpu.async_copy(ent_hbm.at[h_iv], h_rows, sem_h)
    ct = pltpu.async_copy(ent_hbm.at[t_iv], t_rows, sem_t)
    cr = pltpu.async_copy(rel_hbm.at[r_iv], r_rows, sem_r)
    ch.wait()
    pltpu.sync_copy(h_rows, h_out.at[pl.ds(base, _B_PER_W)])
    ct.wait()
    pltpu.sync_copy(t_rows, t_out.at[pl.ds(base, _B_PER_W)])
    cr.wait()
    pltpu.sync_copy(r_rows, r_out.at[pl.ds(base, _B_PER_W)])


_gather3 = functools.partial(
    pl.kernel,
    out_type=[jax.ShapeDtypeStruct((_BATCH, _EMB_DIM), jnp.float32)] * 3,
    mesh=plsc.VectorSubcoreMesh(core_axis_name="c", subcore_axis_name="s"),
    scratch_types=[
        pltpu.VMEM((_B_PER_W,), jnp.int32),
        pltpu.VMEM((_B_PER_W,), jnp.int32),
        pltpu.VMEM((_B_PER_W,), jnp.int32),
        pltpu.VMEM((_B_PER_W, _EMB_DIM), jnp.float32),
        pltpu.VMEM((_B_PER_W, _EMB_DIM), jnp.float32),
        pltpu.VMEM((_B_PER_W, _EMB_DIM), jnp.float32),
        pltpu.SemaphoreType.DMA,
        pltpu.SemaphoreType.DMA,
        pltpu.SemaphoreType.DMA,
    ],
    compiler_params=pltpu.CompilerParams(use_tc_tiling_on_sc=False),
)(_gather_body)


_ROWS2 = _BATCH // 2
_CHUNK = 1024
_NCHUNK = _ROWS2 // _CHUNK


def _score_body(h_ref, t_ref, r_ref, oe_ref, oo_ref):
    def ss_step(i, acc):
        s = pl.ds(i * _CHUNK, _CHUNK)
        hh = h_ref[s, :]
        tt = t_ref[s, :]
        rr = r_ref[s, :]
        hh = hh * hh
        tt = tt * tt
        rr = rr * rr
        return (
            acc[0] + jnp.sum(hh[:, :_HALF]) + jnp.sum(hh[:, 2 * _HALF:3 * _HALF]),
            acc[1] + jnp.sum(hh[:, _HALF:2 * _HALF]) + jnp.sum(hh[:, 3 * _HALF:]),
            acc[2] + jnp.sum(tt[:, :_HALF]) + jnp.sum(tt[:, 2 * _HALF:3 * _HALF]),
            acc[3] + jnp.sum(tt[:, _HALF:2 * _HALF]) + jnp.sum(tt[:, 3 * _HALF:]),
            acc[4] + jnp.sum(rr[:, :_HALF]) + jnp.sum(rr[:, 2 * _HALF:3 * _HALF]),
            acc[5] + jnp.sum(rr[:, _HALF:2 * _HALF]) + jnp.sum(rr[:, 3 * _HALF:]),
        )

    z = jnp.float32(0)
    sh0, sh1, st0, st1, sr0, sr1 = lax.fori_loop(
        0, _NCHUNK, ss_step, (z, z, z, z, z, z))
    d0 = jnp.sqrt(sh0) * jnp.sqrt(sr0) * jnp.sqrt(st0)
    d1 = jnp.sqrt(sh1) * jnp.sqrt(sr1) * jnp.sqrt(st1)

    def sc_step(i, carry):
        s = pl.ds(i * _CHUNK, _CHUNK)
        m = h_ref[s, :] * r_ref[s, :] * t_ref[s, :]
        for ofs, oref in ((0, oe_ref), (2 * _HALF, oo_ref)):
            acc = None
            for half, d in ((0, d0), (1, d1)):
                p = 1.0 - m[:, ofs + half * _HALF:ofs + (half + 1) * _HALF] / d
                w = _HALF
                while w > 1:
                    w //= 2
                    p = p[:, :w] * p[:, w:2 * w]
                score = 1.0 - p[:, 0]
                acc = -score if acc is None else acc - score
            oref[s] = acc
        return carry

    lax.fori_loop(0, _NCHUNK, sc_step, 0)


def kernel(ent_embeddings, rel_embeddings, predict_h, predict_t, predict_r):
    h_idx = predict_h.astype(jnp.int32)
    t_idx = predict_t.astype(jnp.int32)
    r_idx = predict_r.astype(jnp.int32)
    h_rows, t_rows, r_rows = _gather3(
        ent_embeddings, rel_embeddings, h_idx, t_idx, r_idx)
    even, odd = pl.pallas_call(
        _score_body,
        out_shape=[jax.ShapeDtypeStruct((_ROWS2,), jnp.float32)] * 2,
    )(h_rows.reshape(_ROWS2, 2 * _EMB_DIM),
      t_rows.reshape(_ROWS2, 2 * _EMB_DIM),
      r_rows.reshape(_ROWS2, 2 * _EMB_DIM))
    return jnp.stack([even, odd], axis=1).reshape(-1)
